# baseline (speedup 1.0000x reference)
import functools

import jax
import jax.numpy as jnp
from jax import lax
from jax.experimental import pallas as pl
from jax.experimental.pallas import tpu as pltpu
from jax.experimental.pallas import tpu_sc as plsc

VOCAB = 1000
B, T = 1024, 50
NC, NS = 2, 16
NW = NC * NS
NSEG = 5
SEG = VOCAB // NSEG
KT = SEG // 8
NJ = B // 128
NBLK = T * NJ * NSEG
NITER = 64
NPAIR = NITER // 2


@functools.partial(
    pl.kernel,
    out_type=jax.ShapeDtypeStruct((T, VOCAB // 8, NJ, 8, 128), jnp.float32),
    mesh=plsc.VectorSubcoreMesh(core_axis_name="c", subcore_axis_name="s"),
    scratch_types=[
        pltpu.VMEM((128,), jnp.int32),
        pltpu.VMEM((128,), jnp.int32),
        pltpu.VMEM((128, SEG), jnp.float32),
        pltpu.VMEM((128, SEG), jnp.float32),
        pltpu.VMEM((1, KT, 1, 8, 128), jnp.float32),
        pltpu.VMEM((1, KT, 1, 8, 128), jnp.float32),
        pltpu.SemaphoreType.DMA,
        pltpu.SemaphoreType.DMA,
        pltpu.SemaphoreType.DMA,
        pltpu.SemaphoreType.DMA,
        pltpu.SemaphoreType.DMA,
        pltpu.SemaphoreType.DMA,
    ],
    compiler_params=pltpu.CompilerParams(
        use_tc_tiling_on_sc=False, needs_layout_passes=False
    ),
)
def _gather_t(ts_hbm, idx_hbm, out_hbm,
              ib0, ib1, ba0, ba1, bb0, bb1,
              is0, is1, gs0, gs1, ss0, ss1):
    wid = lax.axis_index("s") * NC + lax.axis_index("c")
    iota16 = lax.iota(jnp.int32, 16)

    def params(i):
        m = jnp.minimum(wid + NW * i, NBLK - 1)
        t = m // (NJ * NSEG)
        rem = m - t * (NJ * NSEG)
        j = rem // NSEG
        s = rem - j * NSEG
        return t, j, s

    def idx_src(i):
        t, j, s = params(i)
        return idx_hbm.at[s, t, pl.ds(128 * j, 128)]

    def out_dst(i):
        t, j, s = params(i)
        return out_hbm.at[pl.ds(t, 1), pl.ds(KT * s, KT), pl.ds(j, 1)]

    c_idx = [iota16 + 16 * ci for ci in range(8)]

    def transpose(ba, bb):
        @plsc.parallel_loop(0, KT, unroll=2)
        def tbody(k):
            for r in range(8):
                vv = jnp.broadcast_to(8 * k + r, (16,)).astype(jnp.int32)
                for ci in range(8):
                    vec = plsc.load_gather(ba, [c_idx[ci], vv])
                    bb[0, k, 0, r, pl.ds(16 * ci, 16)] = vec

    pltpu.sync_copy(idx_src(0), ib0)
    pltpu.async_copy(ts_hbm.at[ib0], ba0, gs0)
    pltpu.async_copy(idx_src(1), ib1, is1)

    def pair(p, carry):
        a = 2 * p
        pltpu.make_async_copy(idx_src(a + 1), ib1, is1).wait()
        pltpu.async_copy(ts_hbm.at[ib1], ba1, gs1)
        pltpu.async_copy(idx_src(a + 2), ib0, is0)

        pltpu.make_async_copy(ts_hbm.at[ib0], ba0, gs0).wait()

        @pl.when(p > 0)
        def _():
            pltpu.make_async_copy(bb0, out_dst(a - 2), ss0).wait()

        transpose(ba0, bb0)
        pltpu.async_copy(bb0, out_dst(a), ss0)

        pltpu.make_async_copy(idx_src(a + 2), ib0, is0).wait()
        pltpu.async_copy(ts_hbm.at[ib0], ba0, gs0)
        pltpu.async_copy(idx_src(a + 3), ib1, is1)

        pltpu.make_async_copy(ts_hbm.at[ib1], ba1, gs1).wait()

        @pl.when(p > 0)
        def _():
            pltpu.make_async_copy(bb1, out_dst(a - 1), ss1).wait()

        transpose(ba1, bb1)
        pltpu.async_copy(bb1, out_dst(a + 1), ss1)
        return carry

    lax.fori_loop(0, NPAIR, pair, 0)

    pltpu.make_async_copy(ts_hbm.at[ib0], ba0, gs0).wait()
    pltpu.make_async_copy(idx_src(NITER + 1), ib1, is1).wait()
    pltpu.make_async_copy(bb0, out_dst(NITER - 2), ss0).wait()
    pltpu.make_async_copy(bb1, out_dst(NITER - 1), ss1).wait()


def kernel(idx, targets, token_embedding_table):
    idxT = idx.astype(jnp.int32).T
    seg_off = (VOCAB * jnp.arange(NSEG, dtype=jnp.int32))[:, None, None]
    idx5 = idxT[None] + seg_off
    ts = token_embedding_table.reshape(VOCAB, NSEG, SEG)
    ts = ts.transpose(1, 0, 2).reshape(NSEG * VOCAB, SEG)
    x = _gather_t(ts, idx5)
    return x.transpose(2, 4, 0, 1, 3).reshape(B, T, VOCAB)


# speedup vs baseline: 1.0327x; 1.0327x over previous
import functools

import jax
import jax.numpy as jnp
from jax import lax
from jax.experimental import pallas as pl
from jax.experimental.pallas import tpu as pltpu
from jax.experimental.pallas import tpu_sc as plsc

VOCAB = 1000
B, T = 1024, 50
NC, NS = 2, 16
NW = NC * NS
NSEG = 5
SEG = VOCAB // NSEG
KT = SEG // 8
NJ = B // 128
NBLK = T * NJ * NSEG
NITER = 64
NPAIR = NITER // 2


@functools.partial(
    pl.kernel,
    out_type=jax.ShapeDtypeStruct((T, VOCAB // 8, NJ, 8, 128), jnp.float32),
    mesh=plsc.VectorSubcoreMesh(core_axis_name="c", subcore_axis_name="s"),
    scratch_types=[
        pltpu.VMEM((128,), jnp.int32),
        pltpu.VMEM((128,), jnp.int32),
        pltpu.VMEM((128,), jnp.int32),
        pltpu.VMEM((128,), jnp.int32),
        pltpu.VMEM((128, SEG), jnp.float32),
        pltpu.VMEM((128, SEG), jnp.float32),
        pltpu.VMEM((1, KT, 1, 8, 128), jnp.float32),
        pltpu.VMEM((1, KT, 1, 8, 128), jnp.float32),
        pltpu.SemaphoreType.DMA,
        pltpu.SemaphoreType.DMA,
        pltpu.SemaphoreType.DMA,
        pltpu.SemaphoreType.DMA,
        pltpu.SemaphoreType.DMA,
        pltpu.SemaphoreType.DMA,
    ],
    compiler_params=pltpu.CompilerParams(
        use_tc_tiling_on_sc=False, needs_layout_passes=False
    ),
)
def _gather_t(ts_hbm, idx_hbm, out_hbm,
              ir0, ir1, ib0, ib1, ba0, ba1, bb0, bb1,
              is0, is1, gs0, gs1, ss0, ss1):
    wid = lax.axis_index("s") * NC + lax.axis_index("c")
    iota16 = lax.iota(jnp.int32, 16)

    def params(i):
        m = jnp.minimum(wid + NW * i, NBLK - 1)
        t = m // (NJ * NSEG)
        rem = m - t * (NJ * NSEG)
        j = rem // NSEG
        s = rem - j * NSEG
        return t, j, s

    def idx_src(i):
        t, j, s = params(i)
        return idx_hbm.at[t, pl.ds(128 * j, 128)]

    def seg_idx(i, ir, ib):
        _, _, s = params(i)
        for ci in range(8):
            ib[pl.ds(16 * ci, 16)] = ir[pl.ds(16 * ci, 16)] * 5 + s

    def out_dst(i):
        t, j, s = params(i)
        return out_hbm.at[pl.ds(t, 1), pl.ds(KT * s, KT), pl.ds(j, 1)]

    c_idx = [iota16 + 16 * ci for ci in range(8)]

    def transpose(ba, bb):
        @plsc.parallel_loop(0, KT, unroll=2)
        def tbody(k):
            for r in range(8):
                vv = jnp.broadcast_to(8 * k + r, (16,)).astype(jnp.int32)
                for ci in range(8):
                    vec = plsc.load_gather(ba, [c_idx[ci], vv])
                    bb[0, k, 0, r, pl.ds(16 * ci, 16)] = vec

    pltpu.sync_copy(idx_src(0), ir0)
    seg_idx(0, ir0, ib0)
    pltpu.async_copy(ts_hbm.at[ib0], ba0, gs0)
    pltpu.async_copy(idx_src(1), ir1, is1)

    def pair(p, carry):
        a = 2 * p
        pltpu.make_async_copy(idx_src(a + 1), ir1, is1).wait()
        seg_idx(a + 1, ir1, ib1)
        pltpu.async_copy(ts_hbm.at[ib1], ba1, gs1)
        pltpu.async_copy(idx_src(a + 2), ir0, is0)

        pltpu.make_async_copy(ts_hbm.at[ib0], ba0, gs0).wait()

        @pl.when(p > 0)
        def _():
            pltpu.make_async_copy(bb0, out_dst(a - 2), ss0).wait()

        transpose(ba0, bb0)
        pltpu.async_copy(bb0, out_dst(a), ss0)

        pltpu.make_async_copy(idx_src(a + 2), ir0, is0).wait()
        seg_idx(a + 2, ir0, ib0)
        pltpu.async_copy(ts_hbm.at[ib0], ba0, gs0)
        pltpu.async_copy(idx_src(a + 3), ir1, is1)

        pltpu.make_async_copy(ts_hbm.at[ib1], ba1, gs1).wait()

        @pl.when(p > 0)
        def _():
            pltpu.make_async_copy(bb1, out_dst(a - 1), ss1).wait()

        transpose(ba1, bb1)
        pltpu.async_copy(bb1, out_dst(a + 1), ss1)
        return carry

    lax.fori_loop(0, NPAIR, pair, 0)

    pltpu.make_async_copy(ts_hbm.at[ib0], ba0, gs0).wait()
    pltpu.make_async_copy(idx_src(NITER + 1), ir1, is1).wait()
    pltpu.make_async_copy(bb0, out_dst(NITER - 2), ss0).wait()
    pltpu.make_async_copy(bb1, out_dst(NITER - 1), ss1).wait()


def kernel(idx, targets, token_embedding_table):
    idxT = idx.astype(jnp.int32).T
    ts = token_embedding_table.reshape(NSEG * VOCAB, SEG)
    x = _gather_t(ts, idxT)
    return x.transpose(2, 4, 0, 1, 3).reshape(B, T, VOCAB)


# speedup vs baseline: 1.3535x; 1.3106x over previous
import functools

import jax
import jax.numpy as jnp
from jax import lax
from jax.experimental import pallas as pl
from jax.experimental.pallas import tpu as pltpu
from jax.experimental.pallas import tpu_sc as plsc

VOCAB = 1000
B, T = 1024, 50
NC, NS = 2, 16
NW = NC * NS
NSEG = 5
SEG = VOCAB // NSEG
KT = SEG // 8
NJ = B // 128
NBLK = T * NJ * NSEG
NITER = 63


@functools.partial(
    pl.kernel,
    out_type=jax.ShapeDtypeStruct((T, VOCAB // 8, NJ, 8, 128), jnp.float32),
    mesh=plsc.VectorSubcoreMesh(core_axis_name="c", subcore_axis_name="s"),
    scratch_types=(
        [pltpu.VMEM((128,), jnp.int32)] * 6
        + [pltpu.VMEM((128, SEG), jnp.float32)] * 3
        + [pltpu.VMEM((1, KT, 1, 8, 128), jnp.float32)] * 2
        + [pltpu.SemaphoreType.DMA] * 8
    ),
    compiler_params=pltpu.CompilerParams(
        use_tc_tiling_on_sc=False, needs_layout_passes=False
    ),
)
def _gather_t(ts_hbm, idx_hbm, out_hbm,
              ir0, ir1, ir2, ib0, ib1, ib2, ba0, ba1, ba2, bb0, bb1,
              is0, is1, is2, gs0, gs1, gs2, ss0, ss1):
    wid = lax.axis_index("s") * NC + lax.axis_index("c")
    iota16 = lax.iota(jnp.int32, 16)

    def params(i):
        m = jnp.minimum(wid + NW * i, NBLK - 1)
        t = m // (NJ * NSEG)
        rem = m - t * (NJ * NSEG)
        j = rem // NSEG
        s = rem - j * NSEG
        return t, j, s

    def idx_src(i):
        t, j, s = params(i)
        return idx_hbm.at[t, pl.ds(128 * j, 128)]

    def seg_idx(i, ir, ib):
        _, _, s = params(i)
        for ci in range(8):
            ib[pl.ds(16 * ci, 16)] = ir[pl.ds(16 * ci, 16)] * 5 + s

    def out_dst(i):
        t, j, s = params(i)
        return out_hbm.at[pl.ds(t, 1), pl.ds(KT * s, KT), pl.ds(j, 1)]

    c_idx = [iota16 + 16 * ci for ci in range(8)]

    def transpose(ba, bb):
        @plsc.parallel_loop(0, SEG, unroll=2)
        def tbody(v):
            k = v // 8
            r = v - 8 * k
            vv = jnp.broadcast_to(v, (16,)).astype(jnp.int32)
            for ci in range(8):
                vec = plsc.load_gather(ba, [c_idx[ci], vv])
                bb[0, k, 0, r, pl.ds(16 * ci, 16)] = vec

    ir = [ir0, ir1, ir2]
    ib = [ib0, ib1, ib2]
    ba = [ba0, ba1, ba2]
    bb = [bb0, bb1]
    isem = [is0, is1, is2]
    gsem = [gs0, gs1, gs2]
    ssem = [ss0, ss1]

    def block(i, bi, di):
        pltpu.make_async_copy(ts_hbm.at[ib[bi]], ba[bi], gsem[bi]).wait()
        pltpu.async_copy(idx_src(i + 3), ir[bi], isem[bi])

        @pl.when(i >= 2)
        def _():
            pltpu.make_async_copy(bb[di], out_dst(i - 2), ssem[di]).wait()

        transpose(ba[bi], bb[di])
        pltpu.async_copy(bb[di], out_dst(i), ssem[di])
        pltpu.make_async_copy(idx_src(i + 3), ir[bi], isem[bi]).wait()
        seg_idx(i + 3, ir[bi], ib[bi])
        pltpu.async_copy(ts_hbm.at[ib[bi]], ba[bi], gsem[bi])

    pltpu.sync_copy(idx_src(0), ir0)
    seg_idx(0, ir0, ib0)
    pltpu.async_copy(ts_hbm.at[ib0], ba0, gs0)
    pltpu.async_copy(idx_src(1), ir1, is1)
    pltpu.async_copy(idx_src(2), ir2, is2)
    pltpu.make_async_copy(idx_src(1), ir1, is1).wait()
    seg_idx(1, ir1, ib1)
    pltpu.async_copy(ts_hbm.at[ib1], ba1, gs1)
    pltpu.make_async_copy(idx_src(2), ir2, is2).wait()
    seg_idx(2, ir2, ib2)
    pltpu.async_copy(ts_hbm.at[ib2], ba2, gs2)

    def six(q, carry):
        a = 6 * q
        for u in range(6):
            block(a + u, u % 3, u % 2)
        return carry

    lax.fori_loop(0, (NITER - 3) // 6, six, 0)
    for u in range(3):
        block(NITER - 3 + u, u % 3, u % 2)

    for b in range(3):
        pltpu.make_async_copy(ts_hbm.at[ib[b]], ba[b], gsem[b]).wait()
    pltpu.make_async_copy(bb[0], out_dst(NITER - 1), ss0).wait()
    pltpu.make_async_copy(bb[1], out_dst(NITER - 2), ss1).wait()


def kernel(idx, targets, token_embedding_table):
    idxT = idx.astype(jnp.int32).T
    ts = token_embedding_table.reshape(NSEG * VOCAB, SEG)
    x = _gather_t(ts, idxT)
    return x.transpose(2, 4, 0, 1, 3).reshape(B, T, VOCAB)


# speedup vs baseline: 1.3575x; 1.0029x over previous
import functools

import jax
import jax.numpy as jnp
from jax import lax
from jax.experimental import pallas as pl
from jax.experimental.pallas import tpu as pltpu
from jax.experimental.pallas import tpu_sc as plsc

VOCAB = 1000
B, T = 1024, 50
NC, NS = 2, 16
NW = NC * NS
NSEG = 5
SEG = VOCAB // NSEG
KT = SEG // 8
NJ = B // 128
NBLK = T * NJ * NSEG
NITER = 63


@functools.partial(
    pl.kernel,
    out_type=jax.ShapeDtypeStruct((T, VOCAB // 8, NJ, 8, 128), jnp.float32),
    mesh=plsc.VectorSubcoreMesh(core_axis_name="c", subcore_axis_name="s"),
    scratch_types=(
        [pltpu.VMEM((128,), jnp.int32)] * 6
        + [pltpu.VMEM((128, SEG), jnp.float32)] * 3
        + [pltpu.VMEM((1, KT, 1, 8, 128), jnp.float32)] * 2
        + [pltpu.SemaphoreType.DMA] * 8
    ),
    compiler_params=pltpu.CompilerParams(
        use_tc_tiling_on_sc=False, needs_layout_passes=False
    ),
)
def _gather_t(ts_hbm, idx_hbm, out_hbm,
              ir0, ir1, ir2, ib0, ib1, ib2, ba0, ba1, ba2, bb0, bb1,
              is0, is1, is2, gs0, gs1, gs2, ss0, ss1):
    wid = lax.axis_index("s") * NC + lax.axis_index("c")
    iota16 = lax.iota(jnp.int32, 16)

    def params(i):
        m = jnp.minimum(wid + NW * i, NBLK - 1)
        t = m // (NJ * NSEG)
        rem = m - t * (NJ * NSEG)
        j = rem // NSEG
        s = rem - j * NSEG
        return t, j, s

    def idx_src(i):
        t, j, s = params(i)
        return idx_hbm.at[t, pl.ds(128 * j, 128)]

    def seg_idx(i, ir, ib):
        _, _, s = params(i)
        for ci in range(8):
            ib[pl.ds(16 * ci, 16)] = ir[pl.ds(16 * ci, 16)] * 5 + s

    def out_dst(i):
        t, j, s = params(i)
        return out_hbm.at[pl.ds(t, 1), pl.ds(KT * s, KT), pl.ds(j, 1)]

    c_idx = [iota16 + 16 * ci for ci in range(8)]

    def transpose(ba, bb):
        @plsc.parallel_loop(0, SEG, unroll=4)
        def tbody(v):
            k = v // 8
            r = v - 8 * k
            vv = jnp.broadcast_to(v, (16,)).astype(jnp.int32)
            for ci in range(8):
                vec = plsc.load_gather(ba, [c_idx[ci], vv])
                bb[0, k, 0, r, pl.ds(16 * ci, 16)] = vec

    ir = [ir0, ir1, ir2]
    ib = [ib0, ib1, ib2]
    ba = [ba0, ba1, ba2]
    bb = [bb0, bb1]
    isem = [is0, is1, is2]
    gsem = [gs0, gs1, gs2]
    ssem = [ss0, ss1]

    def block(i, bi, di):
        pltpu.make_async_copy(ts_hbm.at[ib[bi]], ba[bi], gsem[bi]).wait()
        pltpu.async_copy(idx_src(i + 3), ir[bi], isem[bi])

        @pl.when(i >= 2)
        def _():
            pltpu.make_async_copy(bb[di], out_dst(i - 2), ssem[di]).wait()

        transpose(ba[bi], bb[di])
        pltpu.async_copy(bb[di], out_dst(i), ssem[di])
        pltpu.make_async_copy(idx_src(i + 3), ir[bi], isem[bi]).wait()
        seg_idx(i + 3, ir[bi], ib[bi])
        pltpu.async_copy(ts_hbm.at[ib[bi]], ba[bi], gsem[bi])

    pltpu.sync_copy(idx_src(0), ir0)
    seg_idx(0, ir0, ib0)
    pltpu.async_copy(ts_hbm.at[ib0], ba0, gs0)
    pltpu.async_copy(idx_src(1), ir1, is1)
    pltpu.async_copy(idx_src(2), ir2, is2)
    pltpu.make_async_copy(idx_src(1), ir1, is1).wait()
    seg_idx(1, ir1, ib1)
    pltpu.async_copy(ts_hbm.at[ib1], ba1, gs1)
    pltpu.make_async_copy(idx_src(2), ir2, is2).wait()
    seg_idx(2, ir2, ib2)
    pltpu.async_copy(ts_hbm.at[ib2], ba2, gs2)

    def six(q, carry):
        a = 6 * q
        for u in range(6):
            block(a + u, u % 3, u % 2)
        return carry

    lax.fori_loop(0, (NITER - 3) // 6, six, 0)
    for u in range(3):
        block(NITER - 3 + u, u % 3, u % 2)

    for b in range(3):
        pltpu.make_async_copy(ts_hbm.at[ib[b]], ba[b], gsem[b]).wait()
    pltpu.make_async_copy(bb[0], out_dst(NITER - 1), ss0).wait()
    pltpu.make_async_copy(bb[1], out_dst(NITER - 2), ss1).wait()


def kernel(idx, targets, token_embedding_table):
    idxT = idx.astype(jnp.int32).T
    ts = token_embedding_table.reshape(NSEG * VOCAB, SEG)
    x = _gather_t(ts, idxT)
    return x.transpose(2, 4, 0, 1, 3).reshape(B, T, VOCAB)
